# Initial kernel scaffold; baseline (speedup 1.0000x reference)
import jax
import jax.numpy as jnp
from jax.experimental import pallas as pl


def kernel(r_idx, e1_idx, e2_idx, e3_idx, e4_idx, e5_idx, e6_idx, edge_index, E_weight, R_weight):
    raise NotImplementedError("write your pallas kernel here")


# speedup vs baseline: 6.0058x; 6.0058x over previous
import functools
import jax
import jax.numpy as jnp
from jax import lax
from jax.experimental import pallas as pl
from jax.experimental.pallas import tpu as pltpu
from jax.experimental.pallas import tpu_sc as plsc

N_ENT = 10000
N_REL = 200
D = 256
B = 4096
N_EDGES = 160000
DECAY = 0.0001
H = 128
NC, NS = 2, 16
EPT = N_EDGES // NS
CH = 128
NFULL = EPT // CH
TAIL = EPT - NFULL * CH
RPW = 624
RTAIL = N_ENT - NS * RPW
W = NC * NS
BPT = B // W
SHIFTS = (42, 85, 128, 170, 213)
SLICES = ((0, 128), (128, 128), (256, 128), (384, 128), (512, 112))


@functools.cache
def _sc_kernels():
    _mesh = plsc.VectorSubcoreMesh(
        core_axis_name="c", subcore_axis_name="s",
        num_cores=NC, num_subcores=NS)

    def _zero_shared(buf, zeros_hbm, dst_sh, s):
        pltpu.sync_copy(zeros_hbm, buf)
        for (o, n) in SLICES:
            pltpu.sync_copy(buf.at[pl.ds(0, n)],
                            dst_sh.at[pl.ds(s * RPW + o, n)])

        @pl.when(s == 0)
        def _():
            pltpu.sync_copy(buf.at[pl.ds(0, RTAIL)],
                            dst_sh.at[pl.ds(NS * RPW, RTAIL)])

    def _write_shared(buf, src_sh, out_hbm, s, hbm_base):
        for (o, n) in SLICES:
            pltpu.sync_copy(src_sh.at[pl.ds(s * RPW + o, n)],
                            buf.at[pl.ds(0, n)])
            pltpu.sync_copy(buf.at[pl.ds(0, n)],
                            out_hbm.at[pl.ds(hbm_base + s * RPW + o, n)])

        @pl.when(s == 0)
        def _():
            pltpu.sync_copy(src_sh.at[pl.ds(NS * RPW, RTAIL)],
                            buf.at[pl.ds(0, RTAIL)])
            pltpu.sync_copy(buf.at[pl.ds(0, RTAIL)],
                            out_hbm.at[pl.ds(hbm_base + NS * RPW, RTAIL)])

    @functools.partial(
        pl.kernel,
        out_type=jax.ShapeDtypeStruct((2 * N_ENT, H), jnp.float32),
        mesh=_mesh,
        scratch_types=[
            pltpu.VMEM((CH,), jnp.int32),
            pltpu.VMEM((TAIL,), jnp.int32),
            pltpu.VMEM((CH, H), jnp.float32),
            pltpu.VMEM_SHARED((N_ENT, H), jnp.float32),
            pltpu.SemaphoreType.DMA,
        ],
    )
    def _deg(ecat, ones_hbm, zeros_hbm, cntcat, idx_v, tidx_v, buf,
             cnt_sh, sem):
        c = lax.axis_index("c")
        s = lax.axis_index("s")
        _zero_shared(buf, zeros_hbm, cnt_sh, s)
        pltpu.sync_copy(ones_hbm, buf)
        plsc.subcore_barrier()
        base = c * N_EDGES + s * EPT

        def body(j, carry):
            pltpu.sync_copy(ecat.at[pl.ds(base + j * CH, CH)], idx_v)
            pltpu.sync_copy(buf, cnt_sh.at[idx_v], add=True)
            return carry

        lax.fori_loop(0, NFULL, body, 0)
        pltpu.sync_copy(ecat.at[pl.ds(base + NFULL * CH, TAIL)], tidx_v)
        pltpu.sync_copy(buf.at[pl.ds(0, TAIL)], cnt_sh.at[tidx_v], add=True)
        plsc.subcore_barrier()
        _write_shared(buf, cnt_sh, cntcat, s, c * N_ENT)

    @functools.partial(
        pl.kernel,
        out_type=jax.ShapeDtypeStruct((2 * N_ENT, H), jnp.float32),
        mesh=_mesh,
        scratch_types=[
            pltpu.VMEM((CH,), jnp.int32),
            pltpu.VMEM((CH,), jnp.int32),
            pltpu.VMEM((TAIL,), jnp.int32),
            pltpu.VMEM((TAIL,), jnp.int32),
            pltpu.VMEM((CH, H), jnp.float32),
            pltpu.VMEM_SHARED((N_ENT, H), jnp.float32),
            pltpu.SemaphoreType.DMA,
        ],
    )
    def _prop(gidx, dstix, ycat, zeros_hbm, zcat,
              sidx, didx, tsidx, tdidx, rows, acc, sem):
        c = lax.axis_index("c")
        s = lax.axis_index("s")
        _zero_shared(rows, zeros_hbm, acc, s)
        plsc.subcore_barrier()
        ebase = s * EPT
        gbase = c * N_EDGES + ebase

        def body(j, carry):
            pltpu.sync_copy(gidx.at[pl.ds(gbase + j * CH, CH)], sidx)
            pltpu.sync_copy(dstix.at[pl.ds(ebase + j * CH, CH)], didx)
            pltpu.async_copy(ycat.at[sidx], rows, sem).wait()
            pltpu.sync_copy(rows, acc.at[didx], add=True)
            return carry

        lax.fori_loop(0, NFULL, body, 0)
        toff = ebase + NFULL * CH
        pltpu.sync_copy(gidx.at[pl.ds(c * N_EDGES + toff, TAIL)], tsidx)
        pltpu.sync_copy(dstix.at[pl.ds(toff, TAIL)], tdidx)
        pltpu.async_copy(ycat.at[tsidx], rows.at[pl.ds(0, TAIL)], sem).wait()
        pltpu.sync_copy(rows.at[pl.ds(0, TAIL)], acc.at[tdidx], add=True)
        plsc.subcore_barrier()
        _write_shared(rows, acc, zcat, s, c * N_ENT)

    @functools.partial(
        pl.kernel,
        out_type=[
            jax.ShapeDtypeStruct((7 * B, D), jnp.float32),
            jax.ShapeDtypeStruct((4 * B, H), jnp.float32),
        ],
        mesh=_mesh,
        scratch_types=[
            pltpu.VMEM((BPT,), jnp.int32),
            pltpu.VMEM((BPT, D), jnp.float32),
            pltpu.VMEM((BPT, H), jnp.float32),
            pltpu.SemaphoreType.DMA,
        ],
    )
    def _gath(icat, emb, rw, s128, ocat, socat, idx_v, rows, srows, sem):
        c = lax.axis_index("c")
        s = lax.axis_index("s")
        wid = s * NC + c
        for k in range(7):
            off = k * B + wid * BPT
            pltpu.sync_copy(icat.at[pl.ds(off, BPT)], idx_v)
            table = emb if k < 6 else rw
            pltpu.async_copy(table.at[idx_v], rows, sem).wait()
            pltpu.sync_copy(rows, ocat.at[pl.ds(off, BPT)])
        for k in range(4):
            off = k * B + wid * BPT
            pltpu.sync_copy(icat.at[pl.ds(off, BPT)], idx_v)
            pltpu.async_copy(s128.at[idx_v], srows, sem).wait()
            pltpu.sync_copy(srows, socat.at[pl.ds(off, BPT)])

    return _deg, _prop, _gath


_R2 = 1000
_NB2 = N_ENT // _R2
_R8 = 512
_NB8 = B // _R8


def _prep_body(e_ref, cs_ref, cd_ref, yh0_ref, yh1_ref, ab_ref, b_ref,
               s_ref):
    e = e_ref[...]
    a = lax.rsqrt(jnp.maximum(cs_ref[...][:, 0:1], 1.0))
    b = lax.rsqrt(jnp.maximum(cd_ref[...][:, 0:1], 1.0))
    y = e * a
    yh0_ref[...] = y[:, :H]
    yh1_ref[...] = y[:, H:]
    ab_ref[...] = jnp.broadcast_to(a * b, ab_ref.shape)
    b_ref[...] = jnp.broadcast_to(b, b_ref.shape)
    s_ref[...] = jnp.broadcast_to(
        jnp.sum(e * e, axis=1, keepdims=True), s_ref.shape)


_prep = pl.pallas_call(
    _prep_body,
    grid=(_NB2,),
    in_specs=[
        pl.BlockSpec((_R2, D), lambda i: (i, 0)),
        pl.BlockSpec((_R2, H), lambda i: (i, 0)),
        pl.BlockSpec((_R2, H), lambda i: (_NB2 + i, 0)),
    ],
    out_specs=[
        pl.BlockSpec((_R2, H), lambda i: (i, 0)),
        pl.BlockSpec((_R2, H), lambda i: (i, 0)),
        pl.BlockSpec((_R2, 16), lambda i: (i, 0)),
        pl.BlockSpec((_R2, 16), lambda i: (i, 0)),
        pl.BlockSpec((_R2, H), lambda i: (i, 0)),
    ],
    out_shape=[
        jax.ShapeDtypeStruct((N_ENT, H), jnp.float32),
        jax.ShapeDtypeStruct((N_ENT, H), jnp.float32),
        jax.ShapeDtypeStruct((N_ENT, 16), jnp.float32),
        jax.ShapeDtypeStruct((N_ENT, 16), jnp.float32),
        jax.ShapeDtypeStruct((N_ENT, H), jnp.float32),
    ],
)


def _mid_body(z_ref, ab_ref, y_ref):
    y_ref[...] = z_ref[...] * ab_ref[...][:, 0:1]


_mid = pl.pallas_call(
    _mid_body,
    grid=(2 * _NB2,),
    in_specs=[
        pl.BlockSpec((_R2, H), lambda i: (i, 0)),
        pl.BlockSpec((_R2, 16), lambda i: (i % _NB2, 0)),
    ],
    out_specs=pl.BlockSpec((_R2, H), lambda i: (i, 0)),
    out_shape=jax.ShapeDtypeStruct((2 * N_ENT, H), jnp.float32),
)


def _emb_body(e_ref, z10, z11, z20, z21, b_ref, emb_ref):
    b = b_ref[...][:, 0:1]
    zs = jnp.concatenate([z10[...] + z20[...], z11[...] + z21[...]], axis=1)
    emb_ref[...] = (e_ref[...] + b * zs) * (1.0 / 3.0)


_emb = pl.pallas_call(
    _emb_body,
    grid=(_NB2,),
    in_specs=[
        pl.BlockSpec((_R2, D), lambda i: (i, 0)),
        pl.BlockSpec((_R2, H), lambda i: (i, 0)),
        pl.BlockSpec((_R2, H), lambda i: (_NB2 + i, 0)),
        pl.BlockSpec((_R2, H), lambda i: (i, 0)),
        pl.BlockSpec((_R2, H), lambda i: (_NB2 + i, 0)),
        pl.BlockSpec((_R2, 16), lambda i: (i, 0)),
    ],
    out_specs=pl.BlockSpec((_R2, D), lambda i: (i, 0)),
    out_shape=jax.ShapeDtypeStruct((N_ENT, D), jnp.float32),
)


def _fin_body(g1, g2, g3, g4, g5, g6, gr, s1, s2, s3, s4,
              x_ref, r4_ref, rf_ref):
    i = pl.program_id(0)

    def roll(v, sh):
        return jnp.concatenate([v[:, sh:], v[:, :sh]], axis=1)

    p = gr[...] * g1[...]
    p = p * roll(g2[...], SHIFTS[0])
    p = p * roll(g3[...], SHIFTS[1])
    p = p * roll(g4[...], SHIFTS[2])
    p = p * roll(g5[...], SHIFTS[3])
    p = p * roll(g6[...], SHIFTS[4])
    x_ref[...] = jnp.sum(p, axis=1)
    parts = jnp.stack([
        jnp.sum(s1[...][:, 0]), jnp.sum(s2[...][:, 0]),
        jnp.sum(s3[...][:, 0]), jnp.sum(s4[...][:, 0])]).reshape(1, 4)

    @pl.when(i == 0)
    def _():
        r4_ref[...] = jnp.zeros((1, 4), jnp.float32)

    r4_ref[...] += parts

    @pl.when(i == _NB8 - 1)
    def _():
        rf_ref[...] = (DECAY * jnp.sum(jnp.sqrt(r4_ref[...]))).reshape(1, 1)


def _slot_map(k):
    return lambda i: (k * _NB8 + i, 0)


_fin = pl.pallas_call(
    _fin_body,
    grid=(_NB8,),
    in_specs=(
        [pl.BlockSpec((_R8, D), _slot_map(k)) for k in range(7)]
        + [pl.BlockSpec((_R8, H), _slot_map(k)) for k in range(4)]
    ),
    out_specs=[
        pl.BlockSpec((_R8,), lambda i: (i,)),
        pl.BlockSpec((1, 4), lambda i: (0, 0)),
        pl.BlockSpec((1, 1), lambda i: (0, 0)),
    ],
    out_shape=[
        jax.ShapeDtypeStruct((B,), jnp.float32),
        jax.ShapeDtypeStruct((1, 4), jnp.float32),
        jax.ShapeDtypeStruct((1, 1), jnp.float32),
    ],
)


@jax.jit
def kernel(r_idx, e1_idx, e2_idx, e3_idx, e4_idx, e5_idx, e6_idx,
           edge_index, E_weight, R_weight):
    src = edge_index[0].astype(jnp.int32)
    dst = edge_index[1].astype(jnp.int32)
    ecat = jnp.concatenate([src, dst])
    gidx = jnp.concatenate([src, src + N_ENT])
    ones128 = jnp.ones((CH, H), jnp.float32)
    zeros128 = jnp.zeros((CH, H), jnp.float32)

    _deg, _prop, _gath = _sc_kernels()
    cntcat = _deg(ecat, ones128, zeros128)
    yh0, yh1, ab16, b16, s128 = _prep(E_weight, cntcat, cntcat)
    ycat0 = jnp.concatenate([yh0, yh1], axis=0)
    zcat1 = _prop(gidx, dst, ycat0, zeros128)
    ycat1 = _mid(zcat1, ab16)
    zcat2 = _prop(gidx, dst, ycat1, zeros128)
    emb = _emb(E_weight, zcat1, zcat1, zcat2, zcat2, b16)
    icat = jnp.concatenate([
        e1_idx, e2_idx, e3_idx, e4_idx, e5_idx, e6_idx, r_idx]
    ).astype(jnp.int32)
    ocat, socat = _gath(icat, emb, R_weight, s128)
    x, _r4, rf = _fin(*([ocat] * 7), *([socat] * 4))
    return x, rf[0, 0]


# speedup vs baseline: 8.2198x; 1.3687x over previous
import functools
import jax
import jax.numpy as jnp
from jax import lax
from jax.experimental import pallas as pl
from jax.experimental.pallas import tpu as pltpu
from jax.experimental.pallas import tpu_sc as plsc

N_ENT = 10000
N_REL = 200
D = 256
B = 4096
N_EDGES = 160000
DECAY = 0.0001
H = 128
NC, NS = 2, 16
EPT = N_EDGES // NS
CH = 128
NFULL = EPT // CH
TAIL = EPT - NFULL * CH
RPW = 624
RTAIL = N_ENT - NS * RPW
W = NC * NS
BPT = B // W
SHIFTS = (42, 85, 128, 170, 213)
SLICES = ((0, 128), (128, 128), (256, 128), (384, 128), (512, 112))


@functools.cache
def _sc_kernels():
    _mesh = plsc.VectorSubcoreMesh(
        core_axis_name="c", subcore_axis_name="s",
        num_cores=NC, num_subcores=NS)

    def _zero_shared(buf, zeros_hbm, dst_sh, s):
        pltpu.sync_copy(zeros_hbm, buf)
        for (o, n) in SLICES:
            pltpu.sync_copy(buf.at[pl.ds(0, n)],
                            dst_sh.at[pl.ds(s * RPW + o, n)])

        @pl.when(s == 0)
        def _():
            pltpu.sync_copy(buf.at[pl.ds(0, RTAIL)],
                            dst_sh.at[pl.ds(NS * RPW, RTAIL)])

    def _write_shared(buf, src_sh, out_hbm, s, hbm_base):
        for (o, n) in SLICES:
            pltpu.sync_copy(src_sh.at[pl.ds(s * RPW + o, n)],
                            buf.at[pl.ds(0, n)])
            pltpu.sync_copy(buf.at[pl.ds(0, n)],
                            out_hbm.at[pl.ds(hbm_base + s * RPW + o, n)])

        @pl.when(s == 0)
        def _():
            pltpu.sync_copy(src_sh.at[pl.ds(NS * RPW, RTAIL)],
                            buf.at[pl.ds(0, RTAIL)])
            pltpu.sync_copy(buf.at[pl.ds(0, RTAIL)],
                            out_hbm.at[pl.ds(hbm_base + NS * RPW, RTAIL)])

    @functools.partial(
        pl.kernel,
        out_type=jax.ShapeDtypeStruct((2 * N_ENT, H), jnp.float32),
        mesh=_mesh,
        scratch_types=[
            pltpu.VMEM((CH,), jnp.int32),
            pltpu.VMEM((TAIL,), jnp.int32),
            pltpu.VMEM((CH, H), jnp.float32),
            pltpu.VMEM_SHARED((N_ENT, H), jnp.float32),
            pltpu.SemaphoreType.DMA,
        ],
    )
    def _deg(ecat, ones_hbm, zeros_hbm, cntcat, idx_v, tidx_v, buf,
             cnt_sh, sem):
        c = lax.axis_index("c")
        s = lax.axis_index("s")
        _zero_shared(buf, zeros_hbm, cnt_sh, s)
        pltpu.sync_copy(ones_hbm, buf)
        plsc.subcore_barrier()
        base = c * N_EDGES + s * EPT

        def body(j, carry):
            pltpu.sync_copy(ecat.at[pl.ds(base + j * CH, CH)], idx_v)
            pltpu.sync_copy(buf, cnt_sh.at[idx_v], add=True)
            return carry

        lax.fori_loop(0, NFULL, body, 0)
        pltpu.sync_copy(ecat.at[pl.ds(base + NFULL * CH, TAIL)], tidx_v)
        pltpu.sync_copy(buf.at[pl.ds(0, TAIL)], cnt_sh.at[tidx_v], add=True)
        plsc.subcore_barrier()
        _write_shared(buf, cnt_sh, cntcat, s, c * N_ENT)

    @functools.partial(
        pl.kernel,
        out_type=jax.ShapeDtypeStruct((2 * N_ENT, H), jnp.float32),
        mesh=_mesh,
        scratch_types=[
            pltpu.VMEM((CH,), jnp.int32),
            pltpu.VMEM((CH,), jnp.int32),
            pltpu.VMEM((CH,), jnp.int32),
            pltpu.VMEM((CH,), jnp.int32),
            pltpu.VMEM((TAIL,), jnp.int32),
            pltpu.VMEM((TAIL,), jnp.int32),
            pltpu.VMEM((CH, H), jnp.float32),
            pltpu.VMEM((CH, H), jnp.float32),
            pltpu.VMEM_SHARED((N_ENT, H), jnp.float32),
            pltpu.SemaphoreType.DMA,
            pltpu.SemaphoreType.DMA,
        ],
    )
    def _prop(gidx, dstix, ycat, zeros_hbm, zcat,
              sidx0, didx0, sidx1, didx1, tsidx, tdidx, rows0, rows1,
              acc, sem0, sem1):
        c = lax.axis_index("c")
        s = lax.axis_index("s")
        _zero_shared(rows0, zeros_hbm, acc, s)
        plsc.subcore_barrier()
        ebase = s * EPT
        gbase = c * N_EDGES + ebase
        nhalf = NFULL // 2

        def load_start(j, sidx, didx, rows, sem):
            pltpu.sync_copy(gidx.at[pl.ds(gbase + j * CH, CH)], sidx)
            pltpu.sync_copy(dstix.at[pl.ds(ebase + j * CH, CH)], didx)
            pltpu.async_copy(ycat.at[sidx], rows, sem)

        load_start(0, sidx0, didx0, rows0, sem0)
        load_start(1, sidx1, didx1, rows1, sem1)

        def body(j2, carry):
            j = 2 * j2
            pltpu.make_async_copy(ycat.at[sidx0], rows0, sem0).wait()
            pltpu.sync_copy(rows0, acc.at[didx0], add=True)

            @pl.when(j2 < nhalf - 1)
            def _():
                load_start(j + 2, sidx0, didx0, rows0, sem0)

            pltpu.make_async_copy(ycat.at[sidx1], rows1, sem1).wait()
            pltpu.sync_copy(rows1, acc.at[didx1], add=True)

            @pl.when(j2 < nhalf - 1)
            def _():
                load_start(j + 3, sidx1, didx1, rows1, sem1)

            return carry

        lax.fori_loop(0, nhalf, body, 0)
        toff = ebase + NFULL * CH
        pltpu.sync_copy(gidx.at[pl.ds(c * N_EDGES + toff, TAIL)], tsidx)
        pltpu.sync_copy(dstix.at[pl.ds(toff, TAIL)], tdidx)
        pltpu.async_copy(ycat.at[tsidx], rows0.at[pl.ds(0, TAIL)],
                         sem0).wait()
        pltpu.sync_copy(rows0.at[pl.ds(0, TAIL)], acc.at[tdidx], add=True)
        plsc.subcore_barrier()
        _write_shared(rows0, acc, zcat, s, c * N_ENT)

    @functools.partial(
        pl.kernel,
        out_type=[
            jax.ShapeDtypeStruct((7 * B, D), jnp.float32),
            jax.ShapeDtypeStruct((4 * B, H), jnp.float32),
        ],
        mesh=_mesh,
        scratch_types=[
            pltpu.VMEM((BPT,), jnp.int32),
            pltpu.VMEM((BPT,), jnp.int32),
            pltpu.VMEM((BPT, D), jnp.float32),
            pltpu.VMEM((BPT, D), jnp.float32),
            pltpu.VMEM((BPT, H), jnp.float32),
            pltpu.VMEM((BPT, H), jnp.float32),
            pltpu.SemaphoreType.DMA,
            pltpu.SemaphoreType.DMA,
        ],
    )
    def _gath(icat, emb, rw, s128, ocat, socat, idx0, idx1, rows0, rows1,
              srows0, srows1, sem0, sem1):
        c = lax.axis_index("c")
        s = lax.axis_index("s")
        wid = s * NC + c
        idxs = (idx0, idx1)
        rowss = (rows0, rows1)
        srowss = (srows0, srows1)
        sems = (sem0, sem1)
        descs = [None] * 11

        def start(k):
            p = k % 2
            table = (emb, rw)[1 if k == 6 else 0] if k < 7 else s128
            ioff = (k if k < 7 else k - 7) * B + wid * BPT
            pltpu.sync_copy(icat.at[pl.ds(ioff, BPT)], idxs[p])
            dstbuf = rowss[p] if k < 7 else srowss[p]
            descs[k] = pltpu.async_copy(table.at[idxs[p]], dstbuf, sems[p])

        start(0)
        for k in range(11):
            if k + 1 < 11:
                start(k + 1)
            p = k % 2
            descs[k].wait()
            if k < 7:
                off = k * B + wid * BPT
                pltpu.sync_copy(rowss[p], ocat.at[pl.ds(off, BPT)])
            else:
                off = (k - 7) * B + wid * BPT
                pltpu.sync_copy(srowss[p], socat.at[pl.ds(off, BPT)])

    return _deg, _prop, _gath


_R2 = 1000
_NB2 = N_ENT // _R2
_R8 = 512
_NB8 = B // _R8


def _prep_body(e_ref, cs_ref, cd_ref, yh0_ref, yh1_ref, ab_ref, b_ref,
               s_ref):
    e = e_ref[...]
    a = lax.rsqrt(jnp.maximum(cs_ref[...][:, 0:1], 1.0))
    b = lax.rsqrt(jnp.maximum(cd_ref[...][:, 0:1], 1.0))
    y = e * a
    yh0_ref[...] = y[:, :H]
    yh1_ref[...] = y[:, H:]
    ab_ref[...] = jnp.broadcast_to(a * b, ab_ref.shape)
    b_ref[...] = jnp.broadcast_to(b, b_ref.shape)
    s_ref[...] = jnp.broadcast_to(
        jnp.sum(e * e, axis=1, keepdims=True), s_ref.shape)


_prep = pl.pallas_call(
    _prep_body,
    grid=(_NB2,),
    in_specs=[
        pl.BlockSpec((_R2, D), lambda i: (i, 0)),
        pl.BlockSpec((_R2, H), lambda i: (i, 0)),
        pl.BlockSpec((_R2, H), lambda i: (_NB2 + i, 0)),
    ],
    out_specs=[
        pl.BlockSpec((_R2, H), lambda i: (i, 0)),
        pl.BlockSpec((_R2, H), lambda i: (i, 0)),
        pl.BlockSpec((_R2, 16), lambda i: (i, 0)),
        pl.BlockSpec((_R2, 16), lambda i: (i, 0)),
        pl.BlockSpec((_R2, H), lambda i: (i, 0)),
    ],
    out_shape=[
        jax.ShapeDtypeStruct((N_ENT, H), jnp.float32),
        jax.ShapeDtypeStruct((N_ENT, H), jnp.float32),
        jax.ShapeDtypeStruct((N_ENT, 16), jnp.float32),
        jax.ShapeDtypeStruct((N_ENT, 16), jnp.float32),
        jax.ShapeDtypeStruct((N_ENT, H), jnp.float32),
    ],
)


def _mid_body(z_ref, ab_ref, y_ref):
    y_ref[...] = z_ref[...] * ab_ref[...][:, 0:1]


_mid = pl.pallas_call(
    _mid_body,
    grid=(2 * _NB2,),
    in_specs=[
        pl.BlockSpec((_R2, H), lambda i: (i, 0)),
        pl.BlockSpec((_R2, 16), lambda i: (i % _NB2, 0)),
    ],
    out_specs=pl.BlockSpec((_R2, H), lambda i: (i, 0)),
    out_shape=jax.ShapeDtypeStruct((2 * N_ENT, H), jnp.float32),
)


def _emb_body(e_ref, z10, z11, z20, z21, b_ref, emb_ref):
    b = b_ref[...][:, 0:1]
    zs = jnp.concatenate([z10[...] + z20[...], z11[...] + z21[...]], axis=1)
    emb_ref[...] = (e_ref[...] + b * zs) * (1.0 / 3.0)


_emb = pl.pallas_call(
    _emb_body,
    grid=(_NB2,),
    in_specs=[
        pl.BlockSpec((_R2, D), lambda i: (i, 0)),
        pl.BlockSpec((_R2, H), lambda i: (i, 0)),
        pl.BlockSpec((_R2, H), lambda i: (_NB2 + i, 0)),
        pl.BlockSpec((_R2, H), lambda i: (i, 0)),
        pl.BlockSpec((_R2, H), lambda i: (_NB2 + i, 0)),
        pl.BlockSpec((_R2, 16), lambda i: (i, 0)),
    ],
    out_specs=pl.BlockSpec((_R2, D), lambda i: (i, 0)),
    out_shape=jax.ShapeDtypeStruct((N_ENT, D), jnp.float32),
)


def _fin_body(g1, g2, g3, g4, g5, g6, gr, s1, s2, s3, s4,
              x_ref, r4_ref, rf_ref):
    i = pl.program_id(0)

    def roll(v, sh):
        return jnp.concatenate([v[:, sh:], v[:, :sh]], axis=1)

    p = gr[...] * g1[...]
    p = p * roll(g2[...], SHIFTS[0])
    p = p * roll(g3[...], SHIFTS[1])
    p = p * roll(g4[...], SHIFTS[2])
    p = p * roll(g5[...], SHIFTS[3])
    p = p * roll(g6[...], SHIFTS[4])
    x_ref[...] = jnp.sum(p, axis=1)
    parts = jnp.stack([
        jnp.sum(s1[...][:, 0]), jnp.sum(s2[...][:, 0]),
        jnp.sum(s3[...][:, 0]), jnp.sum(s4[...][:, 0])]).reshape(1, 4)

    @pl.when(i == 0)
    def _():
        r4_ref[...] = jnp.zeros((1, 4), jnp.float32)

    r4_ref[...] += parts

    @pl.when(i == _NB8 - 1)
    def _():
        rf_ref[...] = (DECAY * jnp.sum(jnp.sqrt(r4_ref[...]))).reshape(1, 1)


def _slot_map(k):
    return lambda i: (k * _NB8 + i, 0)


_fin = pl.pallas_call(
    _fin_body,
    grid=(_NB8,),
    in_specs=(
        [pl.BlockSpec((_R8, D), _slot_map(k)) for k in range(7)]
        + [pl.BlockSpec((_R8, H), _slot_map(k)) for k in range(4)]
    ),
    out_specs=[
        pl.BlockSpec((_R8,), lambda i: (i,)),
        pl.BlockSpec((1, 4), lambda i: (0, 0)),
        pl.BlockSpec((1, 1), lambda i: (0, 0)),
    ],
    out_shape=[
        jax.ShapeDtypeStruct((B,), jnp.float32),
        jax.ShapeDtypeStruct((1, 4), jnp.float32),
        jax.ShapeDtypeStruct((1, 1), jnp.float32),
    ],
)


@jax.jit
def kernel(r_idx, e1_idx, e2_idx, e3_idx, e4_idx, e5_idx, e6_idx,
           edge_index, E_weight, R_weight):
    src = edge_index[0].astype(jnp.int32)
    dst = edge_index[1].astype(jnp.int32)
    ecat = jnp.concatenate([src, dst])
    gidx = jnp.concatenate([src, src + N_ENT])
    ones128 = jnp.ones((CH, H), jnp.float32)
    zeros128 = jnp.zeros((CH, H), jnp.float32)

    _deg, _prop, _gath = _sc_kernels()
    cntcat = _deg(ecat, ones128, zeros128)
    yh0, yh1, ab16, b16, s128 = _prep(E_weight, cntcat, cntcat)
    ycat0 = jnp.concatenate([yh0, yh1], axis=0)
    zcat1 = _prop(gidx, dst, ycat0, zeros128)
    ycat1 = _mid(zcat1, ab16)
    zcat2 = _prop(gidx, dst, ycat1, zeros128)
    emb = _emb(E_weight, zcat1, zcat1, zcat2, zcat2, b16)
    icat = jnp.concatenate([
        e1_idx, e2_idx, e3_idx, e4_idx, e5_idx, e6_idx, r_idx]
    ).astype(jnp.int32)
    ocat, socat = _gath(icat, emb, R_weight, s128)
    x, _r4, rf = _fin(*([ocat] * 7), *([socat] * 4))
    return x, rf[0, 0]
